# Initial kernel scaffold; baseline (speedup 1.0000x reference)
import jax
import jax.numpy as jnp
from jax.experimental import pallas as pl


def kernel(x, edge_index, W0, b0, W1, b1, W2, b2, W3, b3):
    raise NotImplementedError("write your pallas kernel here")


# speedup vs baseline: 4.1737x; 4.1737x over previous
import functools

import jax
import jax.numpy as jnp
from jax import lax
from jax.experimental import pallas as pl
from jax.experimental.pallas import tpu as pltpu
from jax.experimental.pallas import tpu_sc as plsc

_N = 10000
_E = 320000
_D = 128
_H = 128
_C = 64

_NC = 2
_NS = 16
_NW = _NC * _NS
_K = 128
_ECH = -(-(_E // _NW) // _K)
_EPT = _ECH * _K
_E_PAD = _EPT * _NW
_RPT = 632
_N_PAD = _RPT * _NS
_DW = 128
_BR = 2000


def _deg_body(dst_hbm, ones_hbm, zeros_hbm, out_hbm, dst_v, ones_v, acc):
    cid = lax.axis_index("c")
    sid = lax.axis_index("s")
    wid = cid * _NS + sid
    pltpu.sync_copy(zeros_hbm, acc.at[pl.ds(sid * _RPT, _RPT)])
    pltpu.sync_copy(dst_hbm.at[wid], dst_v)
    pltpu.sync_copy(ones_hbm, ones_v)
    plsc.subcore_barrier()

    def body(c, carry):
        pltpu.sync_copy(ones_v, acc.at[dst_v.at[c]], add=True)
        return carry

    lax.fori_loop(0, _ECH, body, 0)
    plsc.subcore_barrier()
    pltpu.sync_copy(acc.at[pl.ds(sid * _RPT, _RPT)],
                    out_hbm.at[cid, pl.ds(sid * _RPT, _RPT)])


_deg_call = pl.kernel(
    _deg_body,
    out_type=jax.ShapeDtypeStruct((_NC, _N_PAD, _DW), jnp.float32),
    mesh=plsc.VectorSubcoreMesh(core_axis_name="c", subcore_axis_name="s"),
    scratch_types=[
        pltpu.VMEM((_ECH, _K), jnp.int32),
        pltpu.VMEM((_K, _DW), jnp.float32),
        pltpu.VMEM_SHARED((_N_PAD, _DW), jnp.float32),
    ],
)


def _msg_body(h_hbm, src_hbm, dst_hbm, zeros_hbm, out_hbm,
              src_v, dst_v, rows_v, acc, sem):
    cid = lax.axis_index("c")
    sid = lax.axis_index("s")
    wid = cid * _NS + sid
    pltpu.sync_copy(zeros_hbm, acc.at[pl.ds(sid * _RPT, _RPT)])
    pltpu.sync_copy(src_hbm.at[wid], src_v)
    pltpu.sync_copy(dst_hbm.at[wid], dst_v)
    plsc.subcore_barrier()

    def body(c, carry):
        pltpu.async_copy(h_hbm.at[src_v.at[c]], rows_v, sem).wait()
        pltpu.sync_copy(rows_v, acc.at[dst_v.at[c]], add=True)
        return carry

    lax.fori_loop(0, _ECH, body, 0)
    plsc.subcore_barrier()
    pltpu.sync_copy(acc.at[pl.ds(sid * _RPT, _RPT)],
                    out_hbm.at[cid, pl.ds(sid * _RPT, _RPT)])


_msg_call = pl.kernel(
    _msg_body,
    out_type=jax.ShapeDtypeStruct((_NC, _N_PAD, _D), jnp.float32),
    mesh=plsc.VectorSubcoreMesh(core_axis_name="c", subcore_axis_name="s"),
    scratch_types=[
        pltpu.VMEM((_ECH, _K), jnp.int32),
        pltpu.VMEM((_ECH, _K), jnp.int32),
        pltpu.VMEM((_K, _D), jnp.float32),
        pltpu.VMEM_SHARED((_N_PAD, _D), jnp.float32),
        pltpu.SemaphoreType.DMA,
    ],
)


def _prep_body(degp_ref, x_ref, normb_ref, h0_ref):
    p = degp_ref[...]
    deg = p[0, :, 0] + p[1, :, 0]
    norm = lax.rsqrt(jnp.maximum(deg, 1.0))
    nb = jnp.broadcast_to(norm[:, None], (_BR, _D))
    normb_ref[...] = nb
    h0_ref[...] = x_ref[...] * nb


def _prep_call(degp, x):
    return pl.pallas_call(
        _prep_body,
        grid=(_N // _BR,),
        in_specs=[
            pl.BlockSpec((_NC, _BR, _DW), lambda i: (0, i, 0)),
            pl.BlockSpec((_BR, _D), lambda i: (i, 0)),
        ],
        out_specs=[
            pl.BlockSpec((_BR, _D), lambda i: (i, 0)),
            pl.BlockSpec((_BR, _D), lambda i: (i, 0)),
        ],
        out_shape=[
            jax.ShapeDtypeStruct((_N, _D), jnp.float32),
            jax.ShapeDtypeStruct((_N, _D), jnp.float32),
        ],
    )(degp, x)


def _layer_body(mp_ref, normb_ref, w_ref, b_ref, out_ref, *, last):
    p = mp_ref[...]
    m = p[0] + p[1]
    h = m * normb_ref[...]
    y = jnp.dot(h, w_ref[...], preferred_element_type=jnp.float32)
    y = y + b_ref[...][None, :]
    if last:
        out_ref[...] = y
    else:
        out_ref[...] = jnp.maximum(y, 0.0) * normb_ref[...]


def _layer_call(mp, normb, w, b, last):
    wout = w.shape[1]
    return pl.pallas_call(
        functools.partial(_layer_body, last=last),
        grid=(_N // _BR,),
        in_specs=[
            pl.BlockSpec((_NC, _BR, _D), lambda i: (0, i, 0)),
            pl.BlockSpec((_BR, _D), lambda i: (i, 0)),
            pl.BlockSpec((_D, wout), lambda i: (0, 0)),
            pl.BlockSpec((wout,), lambda i: (0,)),
        ],
        out_specs=pl.BlockSpec((_BR, wout), lambda i: (i, 0)),
        out_shape=jax.ShapeDtypeStruct((_N, wout), jnp.float32),
    )(mp, normb, w, b)


def kernel(x, edge_index, W0, b0, W1, b1, W2, b2, W3, b3):
    src = edge_index[0]
    dst = edge_index[1]
    pad = _E_PAD - _E
    src3 = jnp.concatenate([src, jnp.zeros((pad,), jnp.int32)]).reshape(
        _NW, _ECH, _K)
    dst3 = jnp.concatenate([dst, jnp.full((pad,), _N, jnp.int32)]).reshape(
        _NW, _ECH, _K)
    zeros_d = jnp.zeros((_RPT, _DW), jnp.float32)
    ones_d = jnp.ones((_K, _DW), jnp.float32)
    zeros_m = jnp.zeros((_RPT, _D), jnp.float32)

    degp = _deg_call(dst3, ones_d, zeros_d)
    normb, h = _prep_call(degp, x)
    for w, b, last in ((W0, b0, False), (W1, b1, False),
                      (W2, b2, False), (W3, b3, True)):
        mp = _msg_call(h, src3, dst3, zeros_m)
        h = _layer_call(mp, normb, w, b, last)
    return h
